# Initial kernel scaffold; baseline (speedup 1.0000x reference)
import functools

import jax
import jax.numpy as jnp
from jax import lax
from jax.experimental import pallas as pl
from jax.experimental.pallas import tpu as pltpu
from jax.experimental.pallas import tpu_sc as plsc

N = 10000
E = 320000
D_IN = 128
HID = 32
G = 64

NC = 2
NS = 16
NW = NC * NS

CHUNK = 128
CH = 79
EPT = CHUNK * CH
E_PAD = NW * EPT

N_PAD = 10240
RPT = N_PAD // NS

NBLK = 10
BLK = N_PAD // NBLK


def _deg_body(dst_hbm, zeros_hbm, ones_hbm, out_hbm, idx_v, ones_v, deg_sh):
    c = lax.axis_index("c")
    s = lax.axis_index("s")
    wid = c * NS + s
    pltpu.sync_copy(dst_hbm.at[wid], idx_v)
    pltpu.sync_copy(ones_hbm, ones_v)
    pltpu.sync_copy(zeros_hbm.at[pl.ds(s * RPT, RPT)], deg_sh.at[pl.ds(s * RPT, RPT)])
    plsc.subcore_barrier()

    def body(j, carry):
        pltpu.sync_copy(ones_v, deg_sh.at[idx_v.at[j]], add=True)
        return carry

    lax.fori_loop(0, CH, body, 0)
    plsc.subcore_barrier()
    pltpu.sync_copy(deg_sh.at[pl.ds(s * RPT, RPT)], out_hbm.at[c, pl.ds(s * RPT, RPT)])


_deg_kernel = pl.kernel(
    _deg_body,
    out_type=jax.ShapeDtypeStruct((NC, N_PAD), jnp.float32),
    mesh=plsc.VectorSubcoreMesh(core_axis_name="c", subcore_axis_name="s"),
    scratch_types=[
        pltpu.VMEM((CH, CHUNK), jnp.int32),
        pltpu.VMEM((CHUNK,), jnp.float32),
        pltpu.VMEM_SHARED((N_PAD,), jnp.float32),
    ],
)


def _msg_body(src_hbm, dst_hbm, hp_hbm, zeros_hbm, out_hbm,
              sidx_v, didx_v, rows_v, s_sh, sem):
    c = lax.axis_index("c")
    s = lax.axis_index("s")
    wid = c * NS + s
    pltpu.sync_copy(src_hbm.at[wid], sidx_v)
    pltpu.sync_copy(dst_hbm.at[wid], didx_v)
    pltpu.sync_copy(zeros_hbm.at[pl.ds(s * RPT, RPT)], s_sh.at[pl.ds(s * RPT, RPT)])
    plsc.subcore_barrier()

    def body(j, carry):
        pltpu.async_copy(hp_hbm.at[sidx_v.at[j]], rows_v, sem).wait()
        pltpu.sync_copy(rows_v, s_sh.at[didx_v.at[j]], add=True)
        return carry

    lax.fori_loop(0, CH, body, 0)
    plsc.subcore_barrier()
    pltpu.sync_copy(s_sh.at[pl.ds(s * RPT, RPT)], out_hbm.at[c, pl.ds(s * RPT, RPT)])


_msg_kernel = pl.kernel(
    _msg_body,
    out_type=jax.ShapeDtypeStruct((NC, N_PAD, HID), jnp.float32),
    mesh=plsc.VectorSubcoreMesh(core_axis_name="c", subcore_axis_name="s"),
    scratch_types=[
        pltpu.VMEM((CH, CHUNK), jnp.int32),
        pltpu.VMEM((CH, CHUNK), jnp.int32),
        pltpu.VMEM((CHUNK, HID), jnp.float32),
        pltpu.VMEM_SHARED((N_PAD, HID), jnp.float32),
        pltpu.SemaphoreType.DMA,
    ],
)


def _tc1_body(x_ref, w1_ref, degp_ref, hp_ref, dinv_ref):
    deg = degp_ref[0, :] + degp_ref[1, :] + 1.0
    dinv = lax.rsqrt(deg)[:, None]
    h = jnp.dot(x_ref[...], w1_ref[...], preferred_element_type=jnp.float32)
    hp_ref[...] = h * dinv
    dinv_ref[...] = dinv


def _tc1(xp, W1, degp):
    return pl.pallas_call(
        _tc1_body,
        grid=(NBLK,),
        in_specs=[
            pl.BlockSpec((BLK, D_IN), lambda i: (i, 0)),
            pl.BlockSpec((D_IN, HID), lambda i: (0, 0)),
            pl.BlockSpec((NC, BLK), lambda i: (0, i)),
        ],
        out_specs=[
            pl.BlockSpec((BLK, HID), lambda i: (i, 0)),
            pl.BlockSpec((BLK, 1), lambda i: (i, 0)),
        ],
        out_shape=[
            jax.ShapeDtypeStruct((N_PAD, HID), jnp.float32),
            jax.ShapeDtypeStruct((N_PAD, 1), jnp.float32),
        ],
    )(xp, W1, degp)


def _tc2_body(sp_ref, hp_ref, dinv_ref, batch_ref, b1_ref, w2_ref, b2_ref,
              out_ref, sums_sc, cnt_sc):
    i = pl.program_id(0)

    @pl.when(i == 0)
    def _init():
        sums_sc[...] = jnp.zeros_like(sums_sc)
        cnt_sc[...] = jnp.zeros_like(cnt_sc)

    s_tot = sp_ref[0] + sp_ref[1]
    x1 = jnp.maximum(dinv_ref[...] * (s_tot + hp_ref[...]) + b1_ref[...], 0.0)
    b = jnp.reshape(batch_ref[...], (1, BLK))
    onehot = (lax.broadcasted_iota(jnp.int32, (G, BLK), 0) == b).astype(jnp.float32)
    sums_sc[...] += jnp.dot(onehot, x1, preferred_element_type=jnp.float32)
    cnt_sc[...] += jnp.sum(onehot, axis=1, keepdims=True)

    @pl.when(i == NBLK - 1)
    def _final():
        mean = sums_sc[...] / jnp.maximum(cnt_sc[...], 1.0)
        z = jnp.dot(mean, w2_ref[...], preferred_element_type=jnp.float32) + b2_ref[...]
        out_ref[...] = jax.nn.sigmoid(z)


def _tc2(sp, hp, dinv, batch_pad, b1, W2, b2):
    return pl.pallas_call(
        _tc2_body,
        grid=(NBLK,),
        in_specs=[
            pl.BlockSpec((NC, BLK, HID), lambda i: (0, i, 0)),
            pl.BlockSpec((BLK, HID), lambda i: (i, 0)),
            pl.BlockSpec((BLK, 1), lambda i: (i, 0)),
            pl.BlockSpec((BLK,), lambda i: (i,)),
            pl.BlockSpec((HID,), lambda i: (0,)),
            pl.BlockSpec((HID, 1), lambda i: (0, 0)),
            pl.BlockSpec((1,), lambda i: (0,)),
        ],
        out_specs=pl.BlockSpec((G, 1), lambda i: (0, 0)),
        out_shape=jax.ShapeDtypeStruct((G, 1), jnp.float32),
        scratch_shapes=[
            pltpu.VMEM((G, HID), jnp.float32),
            pltpu.VMEM((G, 1), jnp.float32),
        ],
    )(sp, hp, dinv, batch_pad, b1, W2, b2)


def kernel(x, edge_index, batch, W1, b1, W2, b2):
    src = edge_index[0].astype(jnp.int32)
    dst = edge_index[1].astype(jnp.int32)
    pad_idx = jnp.full((E_PAD - E,), N, jnp.int32)
    src3 = jnp.concatenate([src, pad_idx]).reshape(NW, CH, CHUNK)
    dst3 = jnp.concatenate([dst, pad_idx]).reshape(NW, CH, CHUNK)

    xp = jnp.pad(x, ((0, N_PAD - N), (0, 0)))
    batch_pad = jnp.concatenate(
        [batch.astype(jnp.int32), jnp.full((N_PAD - N,), G, jnp.int32)])

    zeros1 = jnp.zeros((N_PAD,), jnp.float32)
    zeros2 = jnp.zeros((N_PAD, HID), jnp.float32)
    ones_c = jnp.ones((CHUNK,), jnp.float32)

    degp = _deg_kernel(dst3, zeros1, ones_c)
    hp, dinv = _tc1(xp, W1, degp)
    sp = _msg_kernel(src3, dst3, hp, zeros2)
    out = _tc2(sp, hp, dinv, batch_pad, b1, W2, b2)
    return out.reshape(-1)


# speedup vs baseline: 39.3264x; 39.3264x over previous
import functools

import jax
import jax.numpy as jnp
from jax import lax
from jax.experimental import pallas as pl
from jax.experimental.pallas import tpu as pltpu
from jax.experimental.pallas import tpu_sc as plsc

N = 10000
E = 320000
D_IN = 128
HID = 32
G = 64

NC = 2
NS = 16
NW = NC * NS

CHUNK = 128
CH = 79
EPT = CHUNK * CH
E_PAD = NW * EPT

N_PAD = 10240
RPT = N_PAD // NS

NBLK = 10
BLK = N_PAD // NBLK


def _deg_body(dst_hbm, zeros_hbm, ones_hbm, out_hbm, idx_v, ones_v, deg_sh):
    c = lax.axis_index("c")
    s = lax.axis_index("s")
    wid = c * NS + s
    pltpu.sync_copy(dst_hbm.at[wid], idx_v)
    pltpu.sync_copy(ones_hbm, ones_v)
    pltpu.sync_copy(zeros_hbm.at[pl.ds(s * RPT, RPT)], deg_sh.at[pl.ds(s * RPT, RPT)])
    plsc.subcore_barrier()

    def body(j, carry):
        pltpu.sync_copy(ones_v, deg_sh.at[idx_v.at[j]], add=True)
        return carry

    lax.fori_loop(0, CH, body, 0)
    plsc.subcore_barrier()
    pltpu.sync_copy(deg_sh.at[pl.ds(s * RPT, RPT)], out_hbm.at[c, pl.ds(s * RPT, RPT)])


_deg_kernel = pl.kernel(
    _deg_body,
    out_type=jax.ShapeDtypeStruct((NC, N_PAD), jnp.float32),
    mesh=plsc.VectorSubcoreMesh(core_axis_name="c", subcore_axis_name="s"),
    scratch_types=[
        pltpu.VMEM((CH, CHUNK), jnp.int32),
        pltpu.VMEM((CHUNK,), jnp.float32),
        pltpu.VMEM_SHARED((N_PAD,), jnp.float32),
    ],
)


def _msg_body(src_hbm, dst_hbm, hp_hbm, zeros_hbm, out_hbm,
              sidx_v, didx_v, rows_v, s_sh, sem):
    c = lax.axis_index("c")
    s = lax.axis_index("s")
    wid = c * NS + s
    pltpu.sync_copy(src_hbm.at[wid], sidx_v)
    pltpu.sync_copy(dst_hbm.at[wid], didx_v)
    pltpu.sync_copy(zeros_hbm.at[pl.ds(s * RPT, RPT)], s_sh.at[pl.ds(s * RPT, RPT)])
    plsc.subcore_barrier()

    def body(j, carry):
        pltpu.async_copy(hp_hbm.at[sidx_v.at[j]], rows_v, sem).wait()
        pltpu.sync_copy(rows_v, s_sh.at[didx_v.at[j]], add=True)
        return carry

    lax.fori_loop(0, CH, body, 0)
    plsc.subcore_barrier()
    pltpu.sync_copy(s_sh.at[pl.ds(s * RPT, RPT)], out_hbm.at[c, pl.ds(s * RPT, RPT)])


_msg_kernel = pl.kernel(
    _msg_body,
    out_type=jax.ShapeDtypeStruct((NC, N_PAD, HID), jnp.float32),
    mesh=plsc.VectorSubcoreMesh(core_axis_name="c", subcore_axis_name="s"),
    scratch_types=[
        pltpu.VMEM((CH, CHUNK), jnp.int32),
        pltpu.VMEM((CH, CHUNK), jnp.int32),
        pltpu.VMEM((CHUNK, HID), jnp.float32),
        pltpu.VMEM_SHARED((N_PAD, HID), jnp.float32),
        pltpu.SemaphoreType.DMA,
    ],
    compiler_params=pltpu.CompilerParams(use_tc_tiling_on_sc=False),
)


def _tc1_body(x_ref, w1_ref, degp_ref, hp_ref, dinv_ref):
    deg = degp_ref[0, :] + degp_ref[1, :] + 1.0
    dinv = lax.rsqrt(deg)[:, None]
    h = jnp.dot(x_ref[...], w1_ref[...], preferred_element_type=jnp.float32)
    hp_ref[...] = h * dinv
    dinv_ref[...] = dinv


def _tc1(xp, W1, degp):
    return pl.pallas_call(
        _tc1_body,
        grid=(NBLK,),
        in_specs=[
            pl.BlockSpec((BLK, D_IN), lambda i: (i, 0)),
            pl.BlockSpec((D_IN, HID), lambda i: (0, 0)),
            pl.BlockSpec((NC, BLK), lambda i: (0, i)),
        ],
        out_specs=[
            pl.BlockSpec((BLK, HID), lambda i: (i, 0)),
            pl.BlockSpec((BLK, 1), lambda i: (i, 0)),
        ],
        out_shape=[
            jax.ShapeDtypeStruct((N_PAD, HID), jnp.float32),
            jax.ShapeDtypeStruct((N_PAD, 1), jnp.float32),
        ],
    )(xp, W1, degp)


def _tc2_body(sp_ref, hp_ref, dinv_ref, batch_ref, b1_ref, w2_ref, b2_ref,
              out_ref, sums_sc, cnt_sc):
    i = pl.program_id(0)

    @pl.when(i == 0)
    def _init():
        sums_sc[...] = jnp.zeros_like(sums_sc)
        cnt_sc[...] = jnp.zeros_like(cnt_sc)

    s_tot = sp_ref[0] + sp_ref[1]
    x1 = jnp.maximum(dinv_ref[...] * (s_tot + hp_ref[...]) + b1_ref[...], 0.0)
    b = jnp.reshape(batch_ref[...], (1, BLK))
    onehot = (lax.broadcasted_iota(jnp.int32, (G, BLK), 0) == b).astype(jnp.float32)
    sums_sc[...] += jnp.dot(onehot, x1, preferred_element_type=jnp.float32)
    cnt_sc[...] += jnp.sum(onehot, axis=1, keepdims=True)

    @pl.when(i == NBLK - 1)
    def _final():
        mean = sums_sc[...] / jnp.maximum(cnt_sc[...], 1.0)
        z = jnp.dot(mean, w2_ref[...], preferred_element_type=jnp.float32) + b2_ref[...]
        out_ref[...] = jax.nn.sigmoid(z)


def _tc2(sp, hp, dinv, batch_pad, b1, W2, b2):
    return pl.pallas_call(
        _tc2_body,
        grid=(NBLK,),
        in_specs=[
            pl.BlockSpec((NC, BLK, HID), lambda i: (0, i, 0)),
            pl.BlockSpec((BLK, HID), lambda i: (i, 0)),
            pl.BlockSpec((BLK, 1), lambda i: (i, 0)),
            pl.BlockSpec((BLK,), lambda i: (i,)),
            pl.BlockSpec((HID,), lambda i: (0,)),
            pl.BlockSpec((HID, 1), lambda i: (0, 0)),
            pl.BlockSpec((1,), lambda i: (0,)),
        ],
        out_specs=pl.BlockSpec((G, 1), lambda i: (0, 0)),
        out_shape=jax.ShapeDtypeStruct((G, 1), jnp.float32),
        scratch_shapes=[
            pltpu.VMEM((G, HID), jnp.float32),
            pltpu.VMEM((G, 1), jnp.float32),
        ],
    )(sp, hp, dinv, batch_pad, b1, W2, b2)


def kernel(x, edge_index, batch, W1, b1, W2, b2):
    src = edge_index[0].astype(jnp.int32)
    dst = edge_index[1].astype(jnp.int32)
    pad_idx = jnp.full((E_PAD - E,), N, jnp.int32)
    src3 = jnp.concatenate([src, pad_idx]).reshape(NW, CH, CHUNK)
    dst3 = jnp.concatenate([dst, pad_idx]).reshape(NW, CH, CHUNK)

    xp = jnp.pad(x, ((0, N_PAD - N), (0, 0)))
    batch_pad = jnp.concatenate(
        [batch.astype(jnp.int32), jnp.full((N_PAD - N,), G, jnp.int32)])

    zeros1 = jnp.zeros((N_PAD,), jnp.float32)
    zeros2 = jnp.zeros((N_PAD, HID), jnp.float32)
    ones_c = jnp.ones((CHUNK,), jnp.float32)

    degp = _deg_kernel(dst3, zeros1, ones_c)
    hp, dinv = _tc1(xp, W1, degp)
    sp = _msg_kernel(src3, dst3, hp, zeros2)
    out = _tc2(sp, hp, dinv, batch_pad, b1, W2, b2)
    return out.reshape(-1)
